# baseline (speedup 1.0000x reference)
import functools

import jax
import jax.numpy as jnp
import numpy as np
from jax import lax
from jax.experimental import pallas as pl
from jax.experimental.pallas import tpu as pltpu
from jax.experimental.pallas import tpu_sc as plsc

N = 10000
E = 160000
D = 256
DE = 256
DA = 256
H = 4
TEMP = 8.0

TS_R = 6104
TS_C = 16384
TS = TS_R * TS_C
SENT = E

NC = 2
NSC = 16
NW = NC * NSC

CH = 5120
NCHF = 40
NCHL = 10

f32 = jnp.float32
i32 = jnp.int32


def _mesh():
  return plsc.VectorSubcoreMesh(core_axis_name="c", subcore_axis_name="s",
                                num_cores=NC, num_subcores=NSC)


def _wid():
  return lax.axis_index("s") * NC + lax.axis_index("c")


def _nch(w):
  return jnp.where(w == NW - 1, NCHL, NCHF)


def _copy_idx(src2d, idxb, w):
  @pl.when(w < NW - 1)
  def _():
    pltpu.sync_copy(src2d.at[pl.ds(w * NCHF, NCHF)], idxb)

  @pl.when(w == NW - 1)
  def _():
    pltpu.sync_copy(src2d.at[pl.ds((NW - 1) * NCHF, NCHL)],
                    idxb.at[pl.ds(0, NCHL)])


def _memset_table():
  def body(out_ref):
    out_ref[...] = jnp.full((56, TS_C), SENT, i32)

  return pl.pallas_call(
      body,
      grid=(TS_R // 56,),
      out_specs=pl.BlockSpec((56, TS_C), lambda j: (j, 0)),
      out_shape=jax.ShapeDtypeStruct((TS_R, TS_C), i32),
  )()


def _sc_pid(row1d, col1d):
  def body(rr, cc, pidp, rpid, rbuf, cbuf, pbuf, qbuf, negbuf):
    w = _wid()
    base = w * CH

    def chunk(j, _):
      off = base + j * 128
      pltpu.sync_copy(rr.at[pl.ds(off, 128)], rbuf)
      pltpu.sync_copy(cc.at[pl.ds(off, 128)], cbuf)

      def step(i, _):
        r = rbuf[pl.ds(i * 16, 16)]
        c = cbuf[pl.ds(i * 16, 16)]
        pbuf[pl.ds(i * 16, 16)] = r * N + c
        qbuf[pl.ds(i * 16, 16)] = c * N + r
        return 0

      lax.fori_loop(0, 8, step, 0)
      pltpu.sync_copy(pbuf, pidp.at[pl.ds(off, 128)])
      pltpu.sync_copy(qbuf, rpid.at[pl.ds(off, 128)])
      return 0

    lax.fori_loop(0, _nch(w), chunk, 0)

    @pl.when(w == NW - 1)
    def _():
      negbuf[...] = jnp.full((16,), -1, i32)
      pltpu.sync_copy(negbuf, pidp.at[pl.ds(E, 16)])

  k = pl.kernel(
      body,
      out_type=(
          jax.ShapeDtypeStruct((E + 16,), i32),
          jax.ShapeDtypeStruct((E,), i32),
      ),
      mesh=_mesh(),
      scratch_types=[
          pltpu.VMEM((128,), i32),
          pltpu.VMEM((128,), i32),
          pltpu.VMEM((128,), i32),
          pltpu.VMEM((128,), i32),
          pltpu.VMEM((16,), i32),
      ],
  )
  return k(row1d, col1d)


def _fill_ids(idsbuf, w):
  base = w * CH

  def step(i, _):
    idsbuf[pl.ds(i * 16, 16)] = base + i * 16 + lax.iota(i32, 16)
    return 0

  lax.fori_loop(0, _nch(w) * 8, step, 0)


def _sc_scatter(table_ref, pid2d):
  def body(p2d, tab, idxb, idsbuf, sem):
    w = _wid()
    _fill_ids(idsbuf, w)
    _copy_idx(p2d, idxb, w)

    def chunk(j, _):
      pltpu.async_copy(idsbuf.at[pl.ds(j * 128, 128)],
                       tab.at[idxb.at[j]], sem).wait()
      return 0

    lax.fori_loop(0, _nch(w), chunk, 0)

  k = pl.kernel(
      body,
      out_type=(),
      mesh=_mesh(),
      scratch_types=[
          pltpu.VMEM((NCHF, 128), i32),
          pltpu.VMEM((CH,), i32),
          pltpu.SemaphoreType.DMA,
      ],
  )
  k(pid2d, table_ref)


def _sc_fix(table_ref, pid2d):
  def body(p2d, tab, idxb, idsbuf, wbuf, mbuf, sem):
    w = _wid()
    _fill_ids(idsbuf, w)
    _copy_idx(p2d, idxb, w)

    def chunk(j, _):
      pltpu.async_copy(tab.at[idxb.at[j]], wbuf, sem).wait()

      def step(kk, _):
        ev = idsbuf[pl.ds(j * 128 + kk * 16, 16)]
        wv = wbuf[pl.ds(kk * 16, 16)]
        mbuf[pl.ds(kk * 16, 16)] = jnp.minimum(ev, wv)
        return 0

      lax.fori_loop(0, 8, step, 0)
      pltpu.async_copy(mbuf, tab.at[idxb.at[j]], sem).wait()
      return 0

    lax.fori_loop(0, _nch(w), chunk, 0)

  k = pl.kernel(
      body,
      out_type=(),
      mesh=_mesh(),
      scratch_types=[
          pltpu.VMEM((NCHF, 128), i32),
          pltpu.VMEM((CH,), i32),
          pltpu.VMEM((128,), i32),
          pltpu.VMEM((128,), i32),
          pltpu.SemaphoreType.DMA,
      ],
  )
  k(pid2d, table_ref)


def _sc_lookup(table_ref, rpid2d, pidp):
  def body(r2d, pp, tab, rev, fnd, idxb, candb, p2b, revb, fndb, sem):
    w = _wid()
    base = w * CH
    _copy_idx(r2d, idxb, w)

    def chunk(j, _):
      pltpu.async_copy(tab.at[idxb.at[j]], candb, sem).wait()
      pltpu.async_copy(pp.at[candb], p2b, sem).wait()

      def step(kk, _):
        rp = idxb[j, pl.ds(kk * 16, 16)]
        cnd = candb[pl.ds(kk * 16, 16)]
        p2 = p2b[pl.ds(kk * 16, 16)]
        eq = jnp.logical_and(p2 == rp, cnd < E)
        revb[pl.ds(kk * 16, 16)] = jnp.where(eq, cnd, 0)
        fndb[pl.ds(kk * 16, 16)] = jnp.where(eq, 1.0, 0.0).astype(f32)
        return 0

      lax.fori_loop(0, 8, step, 0)
      pltpu.sync_copy(revb, rev.at[pl.ds(base + j * 128, 128)])
      pltpu.sync_copy(fndb, fnd.at[pl.ds(base + j * 128, 128)])
      return 0

    lax.fori_loop(0, _nch(w), chunk, 0)

  k = pl.kernel(
      body,
      out_type=(
          jax.ShapeDtypeStruct((E,), i32),
          jax.ShapeDtypeStruct((E,), f32),
      ),
      mesh=_mesh(),
      scratch_types=[
          pltpu.VMEM((NCHF, 128), i32),
          pltpu.VMEM((128,), i32),
          pltpu.VMEM((128,), i32),
          pltpu.VMEM((128,), i32),
          pltpu.VMEM((128,), f32),
          pltpu.SemaphoreType.DMA,
      ],
  )
  return k(rpid2d, pidp, table_ref)


GCH = 64


def _sc_gather(x, pospad, ef, row1d, col1d, rev):
  def body(xh, pp, efh, rr, cc, rv, xr, xc, rvo, pr, pc,
           ib, b256, b128, sem):
    w = _wid()
    base = w * CH
    ng = _nch(w) * (128 // GCH)

    def gpass(idx1d, table, buf, out):
      def chunk(i, _):
        for b in range(2):
          c = i * 2 + b
          off = base + c * GCH
          pltpu.sync_copy(idx1d.at[pl.ds(off, GCH)], ib.at[b])
          pltpu.async_copy(table.at[ib.at[b]], buf.at[b], sem).wait()
          pltpu.async_copy(buf.at[b], out.at[pl.ds(off, GCH)], sem).wait()
        return 0

      lax.fori_loop(0, ng // 2, chunk, 0)

    gpass(rr, xh, b256, xr)
    gpass(cc, xh, b256, xc)
    gpass(rv, efh, b256, rvo)
    gpass(rr, pp, b128, pr)
    gpass(cc, pp, b128, pc)

  k = pl.kernel(
      body,
      out_type=(
          jax.ShapeDtypeStruct((E, 256), f32),
          jax.ShapeDtypeStruct((E, 256), f32),
          jax.ShapeDtypeStruct((E, 256), f32),
          jax.ShapeDtypeStruct((E, 128), f32),
          jax.ShapeDtypeStruct((E, 128), f32),
      ),
      mesh=_mesh(),
      scratch_types=[
          pltpu.VMEM((2, GCH), i32),
          pltpu.VMEM((2, GCH, 256), f32),
          pltpu.VMEM((2, GCH, 128), f32),
          pltpu.SemaphoreType.DMA,
      ],
  )
  return k(x, pospad, ef, row1d, col1d, rev)


BE = 640
NBLK = E // BE


def _tc_edge(XR, XC, PR, PC, EF, RV, fnd, p16, biases):
  def body(xr_r, xc_r, pr_r, pc_r, ef_r, rv_r, fn_r,
           wq, wk, wv, wa1, wa2, w1a, w1b, w1c, w1d, w2,
           bq_r, bk_r, bv_r, ba1_r, ba2_r, be1_r, be2_r,
           wd1, bd1_r, wd2, bd2_r,
           ue_o, pr_o, vt_o):
    xr = xr_r[...]
    posr = pr_r[:, :3]
    xc = xc_r[...]
    posc = pc_r[:, :3]
    ef = ef_r[...]
    rv = rv_r[...] * fn_r[...]

    diff = posr - posc
    dist = jnp.sqrt(jnp.sum(diff * diff, axis=1, keepdims=True) + 1e-12)
    dfeat = jnp.concatenate([diff, dist], axis=1)
    hd = jnp.maximum(jnp.dot(dfeat, wd1[...],
                             preferred_element_type=f32) + bd1_r[...], 0.0)
    dm = jax.nn.sigmoid(jnp.dot(hd, wd2[...],
                                preferred_element_type=f32) + bd2_r[...])

    xrb = xr.astype(jnp.bfloat16)
    xcb = xc.astype(jnp.bfloat16)
    efb = ef.astype(jnp.bfloat16)
    rvb = rv.astype(jnp.bfloat16)

    q = jnp.dot(xrb, wq[...], preferred_element_type=f32) + bq_r[...]
    kk = jnp.dot(efb, wk[...], preferred_element_type=f32) + bk_r[...]
    v = jnp.dot(xcb, wv[...], preferred_element_type=f32) + bv_r[...]

    vparts = []
    for h in range(H):
      qh = q[:, h * 64:(h + 1) * 64]
      kh = kk[:, h * 64:(h + 1) * 64]
      qk = jnp.concatenate([qh, kh], axis=1).astype(jnp.bfloat16)
      a1 = jnp.maximum(jnp.dot(qk, wa1[...],
                               preferred_element_type=f32) + ba1_r[...], 0.0)
      ah = jnp.dot(a1.astype(jnp.bfloat16), wa2[...],
                   preferred_element_type=f32) + ba2_r[...]
      ah = ah * (1.0 / TEMP)
      m = jnp.max(ah, axis=1, keepdims=True)
      ex = jnp.exp(ah - m)
      p = ex / jnp.sum(ex, axis=1, keepdims=True)
      pr_o[:, h * 64:(h + 1) * 64] = p
      vparts.append(p * v[:, h * 64:(h + 1) * 64])

    value = jnp.concatenate(vparts, axis=1) * dm
    vt_o[...] = value.T

    h1 = (jnp.dot(xrb, w1a[...], preferred_element_type=f32) +
          jnp.dot(xcb, w1b[...], preferred_element_type=f32) +
          jnp.dot(efb, w1c[...], preferred_element_type=f32) +
          jnp.dot(rvb, w1d[...], preferred_element_type=f32) + be1_r[...])
    h1 = jnp.maximum(h1, 0.0)
    ue_o[...] = jnp.dot(h1.astype(jnp.bfloat16), w2[...],
                        preferred_element_type=f32) + be2_r[...]

  full = lambda a: pl.BlockSpec(a.shape, lambda j: tuple(0 for _ in a.shape))
  in_specs = [
      pl.BlockSpec((BE, 256), lambda j: (j, 0)),
      pl.BlockSpec((BE, 256), lambda j: (j, 0)),
      pl.BlockSpec((BE, 128), lambda j: (j, 0)),
      pl.BlockSpec((BE, 128), lambda j: (j, 0)),
      pl.BlockSpec((BE, 256), lambda j: (j, 0)),
      pl.BlockSpec((BE, 256), lambda j: (j, 0)),
      pl.BlockSpec((BE, 1), lambda j: (j, 0)),
  ] + [full(w) for w in p16] + [full(b) for b in biases]
  out_specs = [
      pl.BlockSpec((BE, 256), lambda j: (j, 0)),
      pl.BlockSpec((BE, 256), lambda j: (j, 0)),
      pl.BlockSpec((256, BE), lambda j: (0, j)),
  ]
  return pl.pallas_call(
      body,
      grid=(NBLK,),
      in_specs=in_specs,
      out_specs=out_specs,
      out_shape=[
          jax.ShapeDtypeStruct((E, 256), f32),
          jax.ShapeDtypeStruct((E, 256), f32),
          jax.ShapeDtypeStruct((256, E), f32),
      ],
  )(XR, XC, PR, PC, EF, RV, fnd, *p16, *biases)


SCHK = 640
SNC = E // SCHK


def _sc_segmax(valueT, row1d):
  def body(vt, rr, aggf, acc, rowb, valb, sem):
    w = _wid()
    f0 = w * 8
    ninf = jnp.full((16,), -jnp.inf, f32)

    def zstep(i, _):
      acc[pl.ds(i * 16, 16)] = ninf
      return 0

    lax.fori_loop(0, 8 * N // 16, zstep, 0)

    def chunk(ci, _):
      for b in range(2):
        c = ci * 2 + b
        pltpu.sync_copy(rr.at[pl.ds(c * SCHK, SCHK)], rowb.at[b])
        pltpu.sync_copy(vt.at[pl.ds(f0, 8), pl.ds(c * SCHK, SCHK)],
                        valb.at[b])

        def vstep(i, _):
          rows = rowb[b, pl.ds(i * 16, 16)]
          for jj in range(8):
            idxv = rows + jj * N
            val = valb[b, jj, pl.ds(i * 16, 16)]
            cur = plsc.load_gather(acc, [idxv])
            plsc.store_scatter(acc, [idxv], jnp.maximum(cur, val))
            re = plsc.load_gather(acc, [idxv])
            need = val > re
            plsc.store_scatter(acc, [idxv], val, mask=need)
          return 0

        lax.fori_loop(0, SCHK // 16, vstep, 0)
      return 0

    lax.fori_loop(0, SNC // 2, chunk, 0)
    pltpu.sync_copy(acc, aggf.at[pl.ds(w * 8 * N, 8 * N)])

  k = pl.kernel(
      body,
      out_type=jax.ShapeDtypeStruct((256 * N,), f32),
      mesh=_mesh(),
      compiler_params=pltpu.CompilerParams(needs_layout_passes=False),
      scratch_types=[
          pltpu.VMEM((8 * N,), f32),
          pltpu.VMEM((2, SCHK), i32),
          pltpu.VMEM((2, 8, SCHK), f32),
          pltpu.SemaphoreType.DMA,
      ],
  )
  return k(valueT, row1d).reshape(256, N)


def _sc_segsum(ue, idx2d, zin, dirname):
  def body(ueh, ix, zz, osum, ocnt, accum, cnts, idxb, ueb, sbuf, zc, ones,
           sem):
    cidx = lax.axis_index("c")
    s = lax.axis_index("s")

    def zc_step(i, _):
      zc[pl.ds(i * 16, 16)] = jnp.zeros((16,), f32)
      return 0

    lax.fori_loop(0, 1000 // 16, zc_step, 0)

    def ones_step(i, _):
      ones[pl.ds(i * 16, 16)] = jnp.ones((16,), f32)
      return 0

    lax.fori_loop(0, 8, ones_step, 0)

    @pl.when(s < 10)
    def _():
      def zchunk(kofs, _):
        pltpu.sync_copy(zz.at[pl.ds(kofs * 40, 40)], sbuf)
        pltpu.sync_copy(sbuf, accum.at[pl.ds(s * 1000 + kofs * 40, 40)])
        return 0

      lax.fori_loop(0, 25, zchunk, 0)

      @pl.when(cidx == 0)
      def _():
        pltpu.sync_copy(zc.at[pl.ds(0, 1000)], cnts.at[pl.ds(s * 1000, 1000)])

    plsc.subcore_barrier()

    rbase = s * 80
    nch = jnp.where(s == NSC - 1, 50, 80)

    @pl.when(s < NSC - 1)
    def _():
      pltpu.sync_copy(ix.at[pl.ds(s * 80, 80)], idxb)

    @pl.when(s == NSC - 1)
    def _():
      pltpu.sync_copy(ix.at[pl.ds(1200, 50)], idxb.at[pl.ds(0, 50)])

    def chunk(j, _):
      erow = rbase + j
      pltpu.sync_copy(ueh.at[pl.ds(erow * 128, 128), pl.ds(cidx * 128, 128)],
                      ueb)
      pltpu.sync_copy(ueb, accum.at[idxb.at[j]], add=True)

      @pl.when(cidx == 0)
      def _():
        pltpu.sync_copy(ones, cnts.at[idxb.at[j]], add=True)

      return 0

    lax.fori_loop(0, nch, chunk, 0)

    plsc.subcore_barrier()

    @pl.when(s < 10)
    def _():
      def ochunk(kofs, _):
        off = s * 1000 + kofs * 40
        pltpu.sync_copy(accum.at[pl.ds(off, 40)], sbuf)
        pltpu.sync_copy(sbuf, osum.at[pl.ds(off, 40), pl.ds(cidx * 128, 128)])
        return 0

      lax.fori_loop(0, 25, ochunk, 0)

      @pl.when(cidx == 0)
      def _():
        pltpu.sync_copy(cnts.at[pl.ds(s * 1000, 1000)], zc)
        pltpu.sync_copy(zc, ocnt.at[pl.ds(s * 1000, 1000)])

  k = pl.kernel(
      body,
      out_type=(
          jax.ShapeDtypeStruct((N, 256), f32),
          jax.ShapeDtypeStruct((N,), f32),
      ),
      mesh=_mesh(),
      scratch_types=[
          pltpu.VMEM_SHARED((N, 128), f32),
          pltpu.VMEM_SHARED((N,), f32),
          pltpu.VMEM((80, 128), i32),
          pltpu.VMEM((128, 128), f32),
          pltpu.VMEM((40, 128), f32),
          pltpu.VMEM((1000,), f32),
          pltpu.VMEM((128,), f32),
          pltpu.SemaphoreType.DMA,
      ],
      name=f"segsum_{dirname}",
  )
  return k(ue, idx2d, zin)


BN = 1000


def _tc_node(x, agg, osum, isum, ocnt, icnt, pf):
  def body(x_r, ag_r, os_r, is_r, oc_r, ic_r,
           wn1a, wn1b, wn2, weaa, weab, wl1, wl2,
           bn1_r, bn2_r, bea_r, bl1_r, bl2_r, out_o):
    xb = x_r[...].astype(jnp.bfloat16)
    ag = ag_r[...]
    ag = jnp.where(jnp.isfinite(ag), ag, 0.0).astype(jnp.bfloat16)
    un = (jnp.dot(xb, wn1a[...], preferred_element_type=f32) +
          jnp.dot(ag, wn1b[...], preferred_element_type=f32) + bn1_r[...])
    un = jnp.maximum(un, 0.0).astype(jnp.bfloat16)
    un = jnp.dot(un, wn2[...], preferred_element_type=f32) + bn2_r[...]

    om = os_r[...] / jnp.maximum(oc_r[...], 1.0)
    im = is_r[...] / jnp.maximum(ic_r[...], 1.0)
    ea = jax.nn.sigmoid(
        jnp.dot(om.astype(jnp.bfloat16), weaa[...], preferred_element_type=f32)
        + jnp.dot(im.astype(jnp.bfloat16), weab[...],
                  preferred_element_type=f32) + bea_r[...])

    nl = jnp.maximum(
        jnp.dot(un.astype(jnp.bfloat16), wl1[...],
                preferred_element_type=f32) + bl1_r[...], 0.0)
    nl = jnp.dot(nl.astype(jnp.bfloat16), wl2[...],
                 preferred_element_type=f32) + bl2_r[...]
    out_o[...] = nl * ea

  full = lambda a: pl.BlockSpec(a.shape, lambda j: tuple(0 for _ in a.shape))
  in_specs = [
      pl.BlockSpec((BN, 256), lambda j: (j, 0)),
      pl.BlockSpec((BN, 256), lambda j: (j, 0)),
      pl.BlockSpec((BN, 256), lambda j: (j, 0)),
      pl.BlockSpec((BN, 256), lambda j: (j, 0)),
      pl.BlockSpec((BN, 1), lambda j: (j, 0)),
      pl.BlockSpec((BN, 1), lambda j: (j, 0)),
  ] + [full(w) for w in pf]
  return pl.pallas_call(
      body,
      grid=(N // BN,),
      in_specs=in_specs,
      out_specs=pl.BlockSpec((BN, 256), lambda j: (j, 0)),
      out_shape=jax.ShapeDtypeStruct((N, 256), f32),
  )(x, agg, osum, isum, ocnt, icnt, *pf)


def kernel(x, edge_feature, edge_index, node_positions, params):
  p = params
  bf = jnp.bfloat16
  edge_index = edge_index.astype(i32)
  row1d = edge_index[0]
  col1d = edge_index[1]

  pospad = jnp.concatenate(
      [node_positions, jnp.zeros((N, 125), f32)], axis=1)

  table0 = _memset_table().reshape(TS)
  tref = jax.new_ref(table0)
  pidp, rpid = _sc_pid(row1d, col1d)
  pid2d = pidp[:E].reshape(E // 128, 128)
  rpid2d = rpid.reshape(E // 128, 128)
  _sc_scatter(tref, pid2d)
  _sc_fix(tref, pid2d)
  _sc_fix(tref, pid2d)
  rev, fnd = _sc_lookup(tref, rpid2d, pidp)

  XR, XC, RV, PR, PC = _sc_gather(x, pospad, edge_feature, row1d, col1d, rev)

  p16 = (
      p['Wq'].astype(bf), p['Wk'].astype(bf), p['Wv'].astype(bf),
      p['Wa1'].astype(bf), p['Wa2'].astype(bf),
      p['We1'][0:256].astype(bf), p['We1'][256:512].astype(bf),
      p['We1'][512:768].astype(bf), p['We1'][768:1024].astype(bf),
      p['We2'].astype(bf),
  )
  biases = (
      p['bq'].reshape(1, -1), p['bk'].reshape(1, -1), p['bv'].reshape(1, -1),
      p['ba1'].reshape(1, -1), p['ba2'].reshape(1, -1),
      p['be1'].reshape(1, -1), p['be2'].reshape(1, -1),
      p['Wd1'], p['bd1'].reshape(1, -1), p['Wd2'], p['bd2'].reshape(1, -1),
  )
  ue, probf, valueT = _tc_edge(XR, XC, PR, PC, edge_feature, RV,
                               fnd.reshape(E, 1), p16, biases)

  aggT = _sc_segmax(valueT, row1d)
  row2d = row1d.reshape(E // 128, 128)
  col2d = col1d.reshape(E // 128, 128)
  zin = jnp.zeros((1000, 128), f32)
  osum, ocnt = _sc_segsum(ue, row2d, zin, "row")
  isum, icnt = _sc_segsum(ue, col2d, zin, "col")

  pf = (
      p['Wn1'][0:256].astype(bf), p['Wn1'][256:512].astype(bf),
      p['Wn2'].astype(bf),
      p['Wea'][0:256].astype(bf), p['Wea'][256:512].astype(bf),
      p['Wl1'].astype(bf), p['Wl2'].astype(bf),
      p['bn1'].reshape(1, -1), p['bn2'].reshape(1, -1),
      p['bea'].reshape(1, -1), p['bl1'].reshape(1, -1),
      p['bl2'].reshape(1, -1),
  )
  final_node = _tc_node(x, aggT.T, osum, isum,
                        ocnt.reshape(N, 1), icnt.reshape(N, 1), pf)

  return final_node, ue, probf.reshape(E, H, DA // H)


# speedup vs baseline: 1.0622x; 1.0622x over previous
import functools

import jax
import jax.numpy as jnp
import numpy as np
from jax import lax
from jax.experimental import pallas as pl
from jax.experimental.pallas import tpu as pltpu
from jax.experimental.pallas import tpu_sc as plsc

N = 10000
E = 160000
D = 256
DE = 256
DA = 256
H = 4
TEMP = 8.0

TS_R = 6104
TS_C = 16384
TS = TS_R * TS_C
SENT = E

NC = 2
NSC = 16
NW = NC * NSC

CH = 5120
NCHF = 40
NCHL = 10

f32 = jnp.float32
i32 = jnp.int32


def _mesh():
  return plsc.VectorSubcoreMesh(core_axis_name="c", subcore_axis_name="s",
                                num_cores=NC, num_subcores=NSC)


def _wid():
  return lax.axis_index("s") * NC + lax.axis_index("c")


def _nch(w):
  return jnp.where(w == NW - 1, NCHL, NCHF)


def _copy_idx(src2d, idxb, w):
  @pl.when(w < NW - 1)
  def _():
    pltpu.sync_copy(src2d.at[pl.ds(w * NCHF, NCHF)], idxb)

  @pl.when(w == NW - 1)
  def _():
    pltpu.sync_copy(src2d.at[pl.ds((NW - 1) * NCHF, NCHL)],
                    idxb.at[pl.ds(0, NCHL)])


def _memset_table():
  def body(out_ref):
    out_ref[...] = jnp.full((56, TS_C), SENT, i32)

  return pl.pallas_call(
      body,
      grid=(TS_R // 56,),
      out_specs=pl.BlockSpec((56, TS_C), lambda j: (j, 0)),
      out_shape=jax.ShapeDtypeStruct((TS_R, TS_C), i32),
  )()


def _sc_pid(row1d, col1d):
  def body(rr, cc, pidp, rpid, rbuf, cbuf, pbuf, qbuf, negbuf):
    w = _wid()
    base = w * CH

    def chunk(j, _):
      off = base + j * 128
      pltpu.sync_copy(rr.at[pl.ds(off, 128)], rbuf)
      pltpu.sync_copy(cc.at[pl.ds(off, 128)], cbuf)

      def step(i, _):
        r = rbuf[pl.ds(i * 16, 16)]
        c = cbuf[pl.ds(i * 16, 16)]
        pbuf[pl.ds(i * 16, 16)] = r * N + c
        qbuf[pl.ds(i * 16, 16)] = c * N + r
        return 0

      lax.fori_loop(0, 8, step, 0)
      pltpu.sync_copy(pbuf, pidp.at[pl.ds(off, 128)])
      pltpu.sync_copy(qbuf, rpid.at[pl.ds(off, 128)])
      return 0

    lax.fori_loop(0, _nch(w), chunk, 0)

    @pl.when(w == NW - 1)
    def _():
      negbuf[...] = jnp.full((16,), -1, i32)
      pltpu.sync_copy(negbuf, pidp.at[pl.ds(E, 16)])

  k = pl.kernel(
      body,
      out_type=(
          jax.ShapeDtypeStruct((E + 16,), i32),
          jax.ShapeDtypeStruct((E,), i32),
      ),
      mesh=_mesh(),
      scratch_types=[
          pltpu.VMEM((128,), i32),
          pltpu.VMEM((128,), i32),
          pltpu.VMEM((128,), i32),
          pltpu.VMEM((128,), i32),
          pltpu.VMEM((16,), i32),
      ],
  )
  return k(row1d, col1d)


def _fill_ids(idsbuf, w):
  base = w * CH

  def step(i, _):
    idsbuf[pl.ds(i * 16, 16)] = base + i * 16 + lax.iota(i32, 16)
    return 0

  lax.fori_loop(0, _nch(w) * 8, step, 0)


def _sc_scatter(table_ref, pid2d):
  def body(p2d, tab, idxb, idsbuf, sem):
    w = _wid()
    _fill_ids(idsbuf, w)
    _copy_idx(p2d, idxb, w)

    def chunk(j, _):
      pltpu.async_copy(idsbuf.at[pl.ds(j * 128, 128)],
                       tab.at[idxb.at[j]], sem).wait()
      return 0

    lax.fori_loop(0, _nch(w), chunk, 0)

  k = pl.kernel(
      body,
      out_type=(),
      mesh=_mesh(),
      scratch_types=[
          pltpu.VMEM((NCHF, 128), i32),
          pltpu.VMEM((CH,), i32),
          pltpu.SemaphoreType.DMA,
      ],
  )
  k(pid2d, table_ref)


def _sc_fix(table_ref, pid2d):
  def body(p2d, tab, idxb, idsbuf, wbuf, mbuf, sem):
    w = _wid()
    _fill_ids(idsbuf, w)
    _copy_idx(p2d, idxb, w)

    def chunk(j, _):
      pltpu.async_copy(tab.at[idxb.at[j]], wbuf, sem).wait()

      def step(kk, _):
        ev = idsbuf[pl.ds(j * 128 + kk * 16, 16)]
        wv = wbuf[pl.ds(kk * 16, 16)]
        mbuf[pl.ds(kk * 16, 16)] = jnp.minimum(ev, wv)
        return 0

      lax.fori_loop(0, 8, step, 0)
      pltpu.async_copy(mbuf, tab.at[idxb.at[j]], sem).wait()
      return 0

    lax.fori_loop(0, _nch(w), chunk, 0)

  k = pl.kernel(
      body,
      out_type=(),
      mesh=_mesh(),
      scratch_types=[
          pltpu.VMEM((NCHF, 128), i32),
          pltpu.VMEM((CH,), i32),
          pltpu.VMEM((128,), i32),
          pltpu.VMEM((128,), i32),
          pltpu.SemaphoreType.DMA,
      ],
  )
  k(pid2d, table_ref)


def _sc_lookup(table_ref, rpid2d, pidp):
  def body(r2d, pp, tab, rev, fnd, idxb, candb, p2b, revb, fndb, sem):
    w = _wid()
    base = w * CH
    _copy_idx(r2d, idxb, w)

    def chunk(j, _):
      pltpu.async_copy(tab.at[idxb.at[j]], candb, sem).wait()
      pltpu.async_copy(pp.at[candb], p2b, sem).wait()

      def step(kk, _):
        rp = idxb[j, pl.ds(kk * 16, 16)]
        cnd = candb[pl.ds(kk * 16, 16)]
        p2 = p2b[pl.ds(kk * 16, 16)]
        eq = jnp.logical_and(p2 == rp, cnd < E)
        revb[pl.ds(kk * 16, 16)] = jnp.where(eq, cnd, 0)
        fndb[pl.ds(kk * 16, 16)] = jnp.where(eq, 1.0, 0.0).astype(f32)
        return 0

      lax.fori_loop(0, 8, step, 0)
      pltpu.sync_copy(revb, rev.at[pl.ds(base + j * 128, 128)])
      pltpu.sync_copy(fndb, fnd.at[pl.ds(base + j * 128, 128)])
      return 0

    lax.fori_loop(0, _nch(w), chunk, 0)

  k = pl.kernel(
      body,
      out_type=(
          jax.ShapeDtypeStruct((E,), i32),
          jax.ShapeDtypeStruct((E,), f32),
      ),
      mesh=_mesh(),
      scratch_types=[
          pltpu.VMEM((NCHF, 128), i32),
          pltpu.VMEM((128,), i32),
          pltpu.VMEM((128,), i32),
          pltpu.VMEM((128,), i32),
          pltpu.VMEM((128,), f32),
          pltpu.SemaphoreType.DMA,
      ],
  )
  return k(rpid2d, pidp, table_ref)


GCH = 64


def _sc_gather(x, pospad, ef, row1d, col1d, rev):
  def body(xh, pp, efh, rr, cc, rv, xr, xc, rvo, pr, pc,
           ib, b256, b128, s0, s1, s2, s3):
    w = _wid()
    base = w * CH
    nc = _nch(w) * (128 // GCH)
    sems = [s0, s1, s2, s3]

    def load_idx(idx1d):
      @pl.when(w < NW - 1)
      def _():
        pltpu.sync_copy(idx1d.at[pl.ds(w * CH, CH)], ib)

      @pl.when(w == NW - 1)
      def _():
        pltpu.sync_copy(idx1d.at[pl.ds((NW - 1) * CH, 1280)],
                        ib.at[pl.ds(0, 1280)])

    def gpass(idx1d, table, buf, out):
      load_idx(idx1d)
      for b in range(4):
        pltpu.async_copy(table.at[ib.at[pl.ds(b * GCH, GCH)]],
                         buf.at[b], sems[b])

      def grp(i, _):
        for b in range(4):
          c = i * 4 + b
          pltpu.make_async_copy(out.at[pl.ds(0, GCH)], buf.at[b],
                                sems[b]).wait()
          pltpu.sync_copy(buf.at[b], out.at[pl.ds(base + c * GCH, GCH)])

          @pl.when(c + 4 < nc)
          def _(bb=b, cc_=c):
            pltpu.async_copy(table.at[ib.at[pl.ds((cc_ + 4) * GCH, GCH)]],
                             buf.at[bb], sems[bb])

        return 0

      lax.fori_loop(0, nc // 4, grp, 0)

    gpass(rr, xh, b256, xr)
    gpass(cc, xh, b256, xc)
    gpass(rv, efh, b256, rvo)
    gpass(rr, pp, b128, pr)
    gpass(cc, pp, b128, pc)

  k = pl.kernel(
      body,
      out_type=(
          jax.ShapeDtypeStruct((E, 256), f32),
          jax.ShapeDtypeStruct((E, 256), f32),
          jax.ShapeDtypeStruct((E, 256), f32),
          jax.ShapeDtypeStruct((E, 128), f32),
          jax.ShapeDtypeStruct((E, 128), f32),
      ),
      mesh=_mesh(),
      scratch_types=[
          pltpu.VMEM((CH,), i32),
          pltpu.VMEM((4, GCH, 256), f32),
          pltpu.VMEM((4, GCH, 128), f32),
          pltpu.SemaphoreType.DMA,
          pltpu.SemaphoreType.DMA,
          pltpu.SemaphoreType.DMA,
          pltpu.SemaphoreType.DMA,
      ],
  )
  return k(x, pospad, ef, row1d, col1d, rev)


BE = 640
NBLK = E // BE


def _tc_edge(XR, XC, PR, PC, EF, RV, fnd, p16, biases):
  def body(xr_r, xc_r, pr_r, pc_r, ef_r, rv_r, fn_r,
           wq, wk, wv, wa1, wa2, w1a, w1b, w1c, w1d, w2,
           bq_r, bk_r, bv_r, ba1_r, ba2_r, be1_r, be2_r,
           wd1, bd1_r, wd2, bd2_r,
           ue_o, pr_o, vt_o):
    xr = xr_r[...]
    posr = pr_r[:, :3]
    xc = xc_r[...]
    posc = pc_r[:, :3]
    ef = ef_r[...]
    rv = rv_r[...] * fn_r[...]

    diff = posr - posc
    dist = jnp.sqrt(jnp.sum(diff * diff, axis=1, keepdims=True) + 1e-12)
    dfeat = jnp.concatenate([diff, dist], axis=1)
    hd = jnp.maximum(jnp.dot(dfeat, wd1[...],
                             preferred_element_type=f32) + bd1_r[...], 0.0)
    dm = jax.nn.sigmoid(jnp.dot(hd, wd2[...],
                                preferred_element_type=f32) + bd2_r[...])

    xrb = xr.astype(jnp.bfloat16)
    xcb = xc.astype(jnp.bfloat16)
    efb = ef.astype(jnp.bfloat16)
    rvb = rv.astype(jnp.bfloat16)

    q = jnp.dot(xrb, wq[...], preferred_element_type=f32) + bq_r[...]
    kk = jnp.dot(efb, wk[...], preferred_element_type=f32) + bk_r[...]
    v = jnp.dot(xcb, wv[...], preferred_element_type=f32) + bv_r[...]

    vparts = []
    for h in range(H):
      qh = q[:, h * 64:(h + 1) * 64]
      kh = kk[:, h * 64:(h + 1) * 64]
      qk = jnp.concatenate([qh, kh], axis=1).astype(jnp.bfloat16)
      a1 = jnp.maximum(jnp.dot(qk, wa1[...],
                               preferred_element_type=f32) + ba1_r[...], 0.0)
      ah = jnp.dot(a1.astype(jnp.bfloat16), wa2[...],
                   preferred_element_type=f32) + ba2_r[...]
      ah = ah * (1.0 / TEMP)
      m = jnp.max(ah, axis=1, keepdims=True)
      ex = jnp.exp(ah - m)
      p = ex / jnp.sum(ex, axis=1, keepdims=True)
      pr_o[:, h * 64:(h + 1) * 64] = p
      vparts.append(p * v[:, h * 64:(h + 1) * 64])

    value = jnp.concatenate(vparts, axis=1) * dm
    vt_o[...] = value.T

    h1 = (jnp.dot(xrb, w1a[...], preferred_element_type=f32) +
          jnp.dot(xcb, w1b[...], preferred_element_type=f32) +
          jnp.dot(efb, w1c[...], preferred_element_type=f32) +
          jnp.dot(rvb, w1d[...], preferred_element_type=f32) + be1_r[...])
    h1 = jnp.maximum(h1, 0.0)
    ue_o[...] = jnp.dot(h1.astype(jnp.bfloat16), w2[...],
                        preferred_element_type=f32) + be2_r[...]

  full = lambda a: pl.BlockSpec(a.shape, lambda j: tuple(0 for _ in a.shape))
  in_specs = [
      pl.BlockSpec((BE, 256), lambda j: (j, 0)),
      pl.BlockSpec((BE, 256), lambda j: (j, 0)),
      pl.BlockSpec((BE, 128), lambda j: (j, 0)),
      pl.BlockSpec((BE, 128), lambda j: (j, 0)),
      pl.BlockSpec((BE, 256), lambda j: (j, 0)),
      pl.BlockSpec((BE, 256), lambda j: (j, 0)),
      pl.BlockSpec((BE, 1), lambda j: (j, 0)),
  ] + [full(w) for w in p16] + [full(b) for b in biases]
  out_specs = [
      pl.BlockSpec((BE, 256), lambda j: (j, 0)),
      pl.BlockSpec((BE, 256), lambda j: (j, 0)),
      pl.BlockSpec((256, BE), lambda j: (0, j)),
  ]
  return pl.pallas_call(
      body,
      grid=(NBLK,),
      in_specs=in_specs,
      out_specs=out_specs,
      out_shape=[
          jax.ShapeDtypeStruct((E, 256), f32),
          jax.ShapeDtypeStruct((E, 256), f32),
          jax.ShapeDtypeStruct((256, E), f32),
      ],
  )(XR, XC, PR, PC, EF, RV, fnd, *p16, *biases)


SCHK = 640
SNC = E // SCHK


def _sc_segmax(valueT, row1d):
  def body(vt, rr, aggf, acc, rowb, valb, sem, semb):
    w = _wid()
    f0 = w * 8
    sems = [sem, semb]
    ninf = jnp.full((16,), -jnp.inf, f32)

    def zstep(i, _):
      acc[pl.ds(i * 16, 16)] = ninf
      return 0

    lax.fori_loop(0, 8 * N // 16, zstep, 0)

    def issue(c, b):
      pltpu.async_copy(rr.at[pl.ds(c * SCHK, SCHK)], rowb.at[b], sems[b])
      pltpu.async_copy(vt.at[pl.ds(f0, 8), pl.ds(c * SCHK, SCHK)],
                       valb.at[b], sems[b])

    issue(0, 0)

    def chunk(ci, _):
      for b in range(2):
        c = ci * 2 + b

        @pl.when(c + 1 < SNC)
        def _(bb=1 - b, cc_=c):
          issue(cc_ + 1, bb)

        pltpu.make_async_copy(rr.at[pl.ds(0, SCHK)], rowb.at[b],
                              sems[b]).wait()
        pltpu.make_async_copy(vt.at[pl.ds(f0, 8), pl.ds(0, SCHK)],
                              valb.at[b], sems[b]).wait()

        def vstep(i, _):
          rows = rowb[b, pl.ds(i * 16, 16)]
          for jj in range(8):
            idxv = rows + jj * N
            val = valb[b, jj, pl.ds(i * 16, 16)]
            cur = plsc.load_gather(acc, [idxv])
            plsc.store_scatter(acc, [idxv], jnp.maximum(cur, val))
            re = plsc.load_gather(acc, [idxv])
            need = val > re
            plsc.store_scatter(acc, [idxv], val, mask=need)
          return 0

        lax.fori_loop(0, SCHK // 16, vstep, 0)
      return 0

    lax.fori_loop(0, SNC // 2, chunk, 0)
    pltpu.sync_copy(acc, aggf.at[pl.ds(w * 8 * N, 8 * N)])

  k = pl.kernel(
      body,
      out_type=jax.ShapeDtypeStruct((256 * N,), f32),
      mesh=_mesh(),
      compiler_params=pltpu.CompilerParams(needs_layout_passes=False),
      scratch_types=[
          pltpu.VMEM((8 * N,), f32),
          pltpu.VMEM((2, SCHK), i32),
          pltpu.VMEM((2, 8, SCHK), f32),
          pltpu.SemaphoreType.DMA,
          pltpu.SemaphoreType.DMA,
      ],
  )
  return k(valueT, row1d).reshape(256, N)


def _sc_segsum(ue, idx2d, zin, dirname):
  def body(ueh, ix, zz, osum, ocnt, accum, cnts, idxb, ueb, sbuf, zc, ones,
           sem):
    cidx = lax.axis_index("c")
    s = lax.axis_index("s")

    def zc_step(i, _):
      zc[pl.ds(i * 16, 16)] = jnp.zeros((16,), f32)
      return 0

    lax.fori_loop(0, 1000 // 16, zc_step, 0)

    def ones_step(i, _):
      ones[pl.ds(i * 16, 16)] = jnp.ones((16,), f32)
      return 0

    lax.fori_loop(0, 8, ones_step, 0)

    @pl.when(s < 10)
    def _():
      def zchunk(kofs, _):
        pltpu.sync_copy(zz.at[pl.ds(kofs * 40, 40)], sbuf)
        pltpu.sync_copy(sbuf, accum.at[pl.ds(s * 1000 + kofs * 40, 40)])
        return 0

      lax.fori_loop(0, 25, zchunk, 0)

      @pl.when(cidx == 0)
      def _():
        pltpu.sync_copy(zc.at[pl.ds(0, 1000)], cnts.at[pl.ds(s * 1000, 1000)])

    plsc.subcore_barrier()

    rbase = s * 80
    nch = jnp.where(s == NSC - 1, 50, 80)

    @pl.when(s < NSC - 1)
    def _():
      pltpu.sync_copy(ix.at[pl.ds(s * 80, 80)], idxb)

    @pl.when(s == NSC - 1)
    def _():
      pltpu.sync_copy(ix.at[pl.ds(1200, 50)], idxb.at[pl.ds(0, 50)])

    def chunk(j, _):
      erow = rbase + j
      pltpu.sync_copy(ueh.at[pl.ds(erow * 128, 128), pl.ds(cidx * 128, 128)],
                      ueb)
      pltpu.sync_copy(ueb, accum.at[idxb.at[j]], add=True)

      @pl.when(cidx == 0)
      def _():
        pltpu.sync_copy(ones, cnts.at[idxb.at[j]], add=True)

      return 0

    lax.fori_loop(0, nch, chunk, 0)

    plsc.subcore_barrier()

    @pl.when(s < 10)
    def _():
      def ochunk(kofs, _):
        off = s * 1000 + kofs * 40
        pltpu.sync_copy(accum.at[pl.ds(off, 40)], sbuf)
        pltpu.sync_copy(sbuf, osum.at[pl.ds(off, 40), pl.ds(cidx * 128, 128)])
        return 0

      lax.fori_loop(0, 25, ochunk, 0)

      @pl.when(cidx == 0)
      def _():
        pltpu.sync_copy(cnts.at[pl.ds(s * 1000, 1000)], zc)
        pltpu.sync_copy(zc, ocnt.at[pl.ds(s * 1000, 1000)])

  k = pl.kernel(
      body,
      out_type=(
          jax.ShapeDtypeStruct((N, 256), f32),
          jax.ShapeDtypeStruct((N,), f32),
      ),
      mesh=_mesh(),
      scratch_types=[
          pltpu.VMEM_SHARED((N, 128), f32),
          pltpu.VMEM_SHARED((N,), f32),
          pltpu.VMEM((80, 128), i32),
          pltpu.VMEM((128, 128), f32),
          pltpu.VMEM((40, 128), f32),
          pltpu.VMEM((1000,), f32),
          pltpu.VMEM((128,), f32),
          pltpu.SemaphoreType.DMA,
      ],
      name=f"segsum_{dirname}",
  )
  return k(ue, idx2d, zin)


BN = 1000


def _tc_node(x, agg, osum, isum, ocnt, icnt, pf):
  def body(x_r, ag_r, os_r, is_r, oc_r, ic_r,
           wn1a, wn1b, wn2, weaa, weab, wl1, wl2,
           bn1_r, bn2_r, bea_r, bl1_r, bl2_r, out_o):
    xb = x_r[...].astype(jnp.bfloat16)
    ag = ag_r[...]
    ag = jnp.where(jnp.isfinite(ag), ag, 0.0).astype(jnp.bfloat16)
    un = (jnp.dot(xb, wn1a[...], preferred_element_type=f32) +
          jnp.dot(ag, wn1b[...], preferred_element_type=f32) + bn1_r[...])
    un = jnp.maximum(un, 0.0).astype(jnp.bfloat16)
    un = jnp.dot(un, wn2[...], preferred_element_type=f32) + bn2_r[...]

    om = os_r[...] / jnp.maximum(oc_r[...], 1.0)
    im = is_r[...] / jnp.maximum(ic_r[...], 1.0)
    ea = jax.nn.sigmoid(
        jnp.dot(om.astype(jnp.bfloat16), weaa[...], preferred_element_type=f32)
        + jnp.dot(im.astype(jnp.bfloat16), weab[...],
                  preferred_element_type=f32) + bea_r[...])

    nl = jnp.maximum(
        jnp.dot(un.astype(jnp.bfloat16), wl1[...],
                preferred_element_type=f32) + bl1_r[...], 0.0)
    nl = jnp.dot(nl.astype(jnp.bfloat16), wl2[...],
                 preferred_element_type=f32) + bl2_r[...]
    out_o[...] = nl * ea

  full = lambda a: pl.BlockSpec(a.shape, lambda j: tuple(0 for _ in a.shape))
  in_specs = [
      pl.BlockSpec((BN, 256), lambda j: (j, 0)),
      pl.BlockSpec((BN, 256), lambda j: (j, 0)),
      pl.BlockSpec((BN, 256), lambda j: (j, 0)),
      pl.BlockSpec((BN, 256), lambda j: (j, 0)),
      pl.BlockSpec((BN, 1), lambda j: (j, 0)),
      pl.BlockSpec((BN, 1), lambda j: (j, 0)),
  ] + [full(w) for w in pf]
  return pl.pallas_call(
      body,
      grid=(N // BN,),
      in_specs=in_specs,
      out_specs=pl.BlockSpec((BN, 256), lambda j: (j, 0)),
      out_shape=jax.ShapeDtypeStruct((N, 256), f32),
  )(x, agg, osum, isum, ocnt, icnt, *pf)


def kernel(x, edge_feature, edge_index, node_positions, params):
  p = params
  bf = jnp.bfloat16
  edge_index = edge_index.astype(i32)
  row1d = edge_index[0]
  col1d = edge_index[1]

  pospad = jnp.concatenate(
      [node_positions, jnp.zeros((N, 125), f32)], axis=1)

  table0 = _memset_table().reshape(TS)
  tref = jax.new_ref(table0)
  pidp, rpid = _sc_pid(row1d, col1d)
  pid2d = pidp[:E].reshape(E // 128, 128)
  rpid2d = rpid.reshape(E // 128, 128)
  _sc_scatter(tref, pid2d)
  _sc_fix(tref, pid2d)
  _sc_fix(tref, pid2d)
  rev, fnd = _sc_lookup(tref, rpid2d, pidp)

  XR, XC, RV, PR, PC = _sc_gather(x, pospad, edge_feature, row1d, col1d, rev)

  p16 = (
      p['Wq'].astype(bf), p['Wk'].astype(bf), p['Wv'].astype(bf),
      p['Wa1'].astype(bf), p['Wa2'].astype(bf),
      p['We1'][0:256].astype(bf), p['We1'][256:512].astype(bf),
      p['We1'][512:768].astype(bf), p['We1'][768:1024].astype(bf),
      p['We2'].astype(bf),
  )
  biases = (
      p['bq'].reshape(1, -1), p['bk'].reshape(1, -1), p['bv'].reshape(1, -1),
      p['ba1'].reshape(1, -1), p['ba2'].reshape(1, -1),
      p['be1'].reshape(1, -1), p['be2'].reshape(1, -1),
      p['Wd1'], p['bd1'].reshape(1, -1), p['Wd2'], p['bd2'].reshape(1, -1),
  )
  ue, probf, valueT = _tc_edge(XR, XC, PR, PC, edge_feature, RV,
                               fnd.reshape(E, 1), p16, biases)

  aggT = _sc_segmax(valueT, row1d)
  row2d = row1d.reshape(E // 128, 128)
  col2d = col1d.reshape(E // 128, 128)
  zin = jnp.zeros((1000, 128), f32)
  osum, ocnt = _sc_segsum(ue, row2d, zin, "row")
  isum, icnt = _sc_segsum(ue, col2d, zin, "col")

  pf = (
      p['Wn1'][0:256].astype(bf), p['Wn1'][256:512].astype(bf),
      p['Wn2'].astype(bf),
      p['Wea'][0:256].astype(bf), p['Wea'][256:512].astype(bf),
      p['Wl1'].astype(bf), p['Wl2'].astype(bf),
      p['bn1'].reshape(1, -1), p['bn2'].reshape(1, -1),
      p['bea'].reshape(1, -1), p['bl1'].reshape(1, -1),
      p['bl2'].reshape(1, -1),
  )
  final_node = _tc_node(x, aggT.T, osum, isum,
                        ocnt.reshape(N, 1), icnt.reshape(N, 1), pf)

  return final_node, ue, probf.reshape(E, H, DA // H)
